# baseline (speedup 1.0000x reference)
import functools

import jax
import jax.numpy as jnp
from jax import lax
from jax.experimental import pallas as pl
from jax.experimental.pallas import tpu as pltpu
from jax.experimental.pallas import tpu_sc as plsc

_KE = 14.399645
_CUTON = 2.5
_SW_CUTOFF = 7.5
_LR_CUTOFF = 10.0
_CUTOFF = 10.0
_HALF_KE = 0.5 * _KE
_INV_SW_WIDTH = 1.0 / (_SW_CUTOFF - _CUTON)
_INV_LR = 1.0 / _LR_CUTOFF
_INV_LR2 = 1.0 / (_LR_CUTOFF * _LR_CUTOFF)
_INV_CUT = 1.0 / _CUTOFF
_INV_A_CONST = 1.0 / (0.8854 * 0.529177)

_NC = 2
_NS = 16
_NW = _NC * _NS
_LANES = 16
_SUBLEN = 128
_CHUNK = 1024
_NSUB = _CHUNK // _SUBLEN


def _rsqrt(x):
    i = lax.bitcast_convert_type(x, jnp.int32)
    i = jnp.int32(0x5F3759DF) - lax.shift_right_logical(i, 1)
    y = lax.bitcast_convert_type(i, jnp.float32)
    xh = 0.5 * x
    for _ in range(2):
        y = y * (1.5 - xh * y * y)
    return y


def _rcp(x):
    i = lax.bitcast_convert_type(x, jnp.int32)
    i = jnp.int32(0x7EF311C3) - i
    y = lax.bitcast_convert_type(i, jnp.float32)
    for _ in range(2):
        y = y * (2.0 - x * y)
    return y


def _pair_energy(xi, yi_, zi_, qi, c6i, zfi, z23i,
                 xj, yj_, zj_, qj, c6j, zfj, z23j):
    dx = xj - xi
    dy = yj_ - yi_
    dz = zj_ - zi_
    r2 = dx * dx + dy * dy + dz * dz
    u = _rsqrt(r2)
    r = r2 * u
    damped = _rsqrt(r2 + 1.0)
    xs = jnp.clip((r - _CUTON) * _INV_SW_WIDTH, 0.0, 1.0)
    s = (xs * xs * xs) * (xs * (6.0 * xs - 15.0) + 10.0)
    coul = damped + s * (u - damped)
    in_lr = r < _LR_CUTOFF
    shifted = jnp.where(in_lr, coul + r * _INV_LR2 - 2.0 * _INV_LR, 0.0)
    e_c = (qi * qj) * shifted
    xc = r * _INV_CUT
    fc = jnp.where(in_lr,
                   1.0 - (xc * xc * xc) * (xc * (6.0 * xc - 15.0) + 10.0),
                   0.0)
    c6ij = c6i * c6j
    r6 = r2 * r2 * r2
    e_d = (-0.5) * c6ij * _rcp(r6 + 1.0) * fc
    inv_a = (z23i + z23j + 1e-9) * _INV_A_CONST
    xz = r * inv_a
    phi = (0.18175 * jnp.exp(-3.19980 * xz)
           + 0.50986 * jnp.exp(-0.94229 * xz)
           + 0.28022 * jnp.exp(-0.40290 * xz)
           + 0.02817 * jnp.exp(-0.20162 * xz))
    e_z = (zfi * zfj) * u * phi * fc
    return e_c + e_z + e_d


def _make_sc_kernel(nt, nchunk):
    rows_per_tile = nt // _NS
    mesh = plsc.VectorSubcoreMesh(core_axis_name="c", subcore_axis_name="s",
                                  num_cores=_NC, num_subcores=_NS)

    nbuf = 2
    scr = []
    for _ in range(nbuf):
        scr += [
            pltpu.VMEM((_NSUB, _SUBLEN), jnp.int32),
            pltpu.VMEM((_NSUB, _SUBLEN), jnp.int32),
            pltpu.VMEM((_CHUNK, 8), jnp.float32),
            pltpu.VMEM((_CHUNK, 8), jnp.float32),
            pltpu.VMEM((_CHUNK,), jnp.float32),
            pltpu.VMEM((_NSUB, _SUBLEN), jnp.int32),
            pltpu.SemaphoreType.DMA,
            pltpu.SemaphoreType.DMA,
            pltpu.SemaphoreType.DMA,
        ]
    scr += [
        pltpu.VMEM((nt // _NS,), jnp.float32),
        pltpu.VMEM_SHARED((nt,), jnp.float32),
    ]

    @functools.partial(
        pl.kernel,
        out_type=jax.ShapeDtypeStruct((_NC, _NS, rows_per_tile), jnp.float32),
        mesh=mesh,
        scratch_types=scr,
        compiler_params=pltpu.CompilerParams(needs_layout_passes=False,
                                             use_tc_tiling_on_sc=False),
    )
    def sc_kernel(table_hbm, idxi_hbm, idxj_hbm, out_hbm, *bufs):
        buf = [bufs[9 * b:9 * b + 9] for b in range(nbuf)]
        zbuf, acc_sp = bufs[9 * nbuf], bufs[9 * nbuf + 1]
        cid = lax.axis_index("c")
        tid = lax.axis_index("s")

        r0 = tid * rows_per_tile

        def zero_body(i, _):
            zbuf[pl.ds(i * _LANES, _LANES)] = jnp.zeros((_LANES,), jnp.float32)
            return 0
        lax.fori_loop(0, rows_per_tile // _LANES, zero_body, 0)
        pltpu.sync_copy(zbuf, acc_sp.at[pl.ds(r0, rows_per_tile)])
        plsc.subcore_barrier()

        wid = cid * _NS + tid
        base_row = wid * (nchunk * _NSUB)
        lanes = lax.iota(jnp.int32, _LANES)

        def idx_start(b, c):
            row0 = base_row + c * _NSUB
            isem = buf[b][6]
            pltpu.async_copy(idxi_hbm.at[pl.ds(row0, _NSUB)], buf[b][0], isem)
            pltpu.async_copy(idxj_hbm.at[pl.ds(row0, _NSUB)], buf[b][1], isem)

        def idx_wait(b):
            isem = buf[b][6]
            pltpu.make_async_copy(idxi_hbm.at[pl.ds(0, _NSUB)],
                                  buf[b][0], isem).wait()
            pltpu.make_async_copy(idxj_hbm.at[pl.ds(0, _NSUB)],
                                  buf[b][1], isem).wait()

        def gather_start(b):
            idxi_v, idxj_v, rows_i, rows_j = buf[b][:4]
            gsem = buf[b][7]
            for sub in range(_NSUB):
                dst = pl.ds(sub * _SUBLEN, _SUBLEN)
                pltpu.async_copy(table_hbm.at[idxi_v.at[sub]],
                                 rows_i.at[dst], gsem)
                pltpu.async_copy(table_hbm.at[idxj_v.at[sub]],
                                 rows_j.at[dst], gsem)

        def gather_wait(b):
            idxi_v, idxj_v, rows_i, rows_j = buf[b][:4]
            gsem = buf[b][7]
            for sub in range(_NSUB):
                dst = pl.ds(sub * _SUBLEN, _SUBLEN)
                pltpu.make_async_copy(table_hbm.at[idxi_v.at[sub]],
                                      rows_i.at[dst], gsem).wait()
                pltpu.make_async_copy(table_hbm.at[idxj_v.at[sub]],
                                      rows_j.at[dst], gsem).wait()

        def idx_snapshot(b):
            idxi_v, sidx = buf[b][0], buf[b][5]

            def snap_row(r, _):
                for kk in range(_SUBLEN // _LANES):
                    sl = pl.ds(kk * _LANES, _LANES)
                    sidx[r, sl] = idxi_v[r, sl]
                return 0

            lax.fori_loop(0, _NSUB, snap_row, 0)

        def compute(b):
            rows_i, rows_j, evals = buf[b][2], buf[b][3], buf[b][4]

            def pair_body(k, _):
                row16 = k * _LANES + lanes

                def col(ref, ci):
                    return plsc.load_gather(
                        ref, [row16, jnp.full((_LANES,), ci, jnp.int32)])

                e16 = _pair_energy(
                    col(rows_i, 0), col(rows_i, 1), col(rows_i, 2),
                    col(rows_i, 3), col(rows_i, 4), col(rows_i, 5),
                    col(rows_i, 6),
                    col(rows_j, 0), col(rows_j, 1), col(rows_j, 2),
                    col(rows_j, 3), col(rows_j, 4), col(rows_j, 5),
                    col(rows_j, 6))
                evals[pl.ds(k * _LANES, _LANES)] = e16
                return 0

            lax.fori_loop(0, _CHUNK // _LANES, pair_body, 0)

        def scatter_start(b):
            evals, sidx, ssem = buf[b][4], buf[b][5], buf[b][8]
            for sub in range(_NSUB):
                pltpu.async_copy(evals.at[pl.ds(sub * _SUBLEN, _SUBLEN)],
                                 acc_sp.at[sidx.at[sub]], ssem, add=True)

        def scatter_drain(b):
            evals, sidx, ssem = buf[b][4], buf[b][5], buf[b][8]
            for sub in range(_NSUB):
                pltpu.make_async_copy(
                    evals.at[pl.ds(sub * _SUBLEN, _SUBLEN)],
                    acc_sp.at[sidx.at[sub]], ssem).wait()

        idx_start(0, 0)
        idx_wait(0)
        gather_start(0)
        idx_start(1, 1)

        def half(b, ob, g, c):
            gather_wait(b)

            @pl.when(g >= 1)
            def _():
                scatter_drain(b)

            idx_snapshot(b)

            @pl.when(c + 2 < nchunk)
            def _():
                idx_start(b, c + 2)

            @pl.when(c + 1 < nchunk)
            def _():
                idx_wait(ob)
                gather_start(ob)

            compute(b)
            scatter_start(b)

        def sched_body(g, _):
            half(0, 1, g, 2 * g)
            half(1, 0, g, 2 * g + 1)
            return 0

        lax.fori_loop(0, nchunk // 2, sched_body, 0)
        scatter_drain(0)
        scatter_drain(1)
        plsc.subcore_barrier()
        pltpu.sync_copy(acc_sp.at[pl.ds(r0, rows_per_tile)],
                        out_hbm.at[cid, tid])

    return sc_kernel


def _combine_body(p_ref, y_ref, o_ref):
    o_ref[...] = p_ref[0] + p_ref[1] + y_ref[...]


def kernel(yi, R, partial_charges, c6_table, Z, idx_m, idx_i, idx_j):
    n = Z.shape[0]
    p = idx_i.shape[0]

    nt = ((n + 1 + 2047) // 2048) * 2048
    pairs_per_sweep = _NW * _CHUNK
    nchunk = (p + pairs_per_sweep - 1) // pairs_per_sweep
    nchunk = ((nchunk + 3) // 4) * 4
    p_pad = nchunk * pairs_per_sweep

    Z = Z.astype(jnp.int32)
    zf = Z.astype(jnp.float32)
    sc6 = jnp.sqrt(jax.nn.softplus(c6_table.astype(jnp.float32)))
    onehot = (Z[:, None] == jnp.arange(sc6.shape[0], dtype=jnp.int32)[None, :])
    sc6z = jnp.dot(onehot.astype(jnp.float32), sc6)
    sqke = _HALF_KE ** 0.5
    table = jnp.stack(
        [R[:, 0], R[:, 1], R[:, 2],
         partial_charges.astype(jnp.float32) * sqke,
         sc6z, zf * sqke, zf ** 0.23, jnp.zeros((n,), jnp.float32)], axis=1)
    table = jnp.pad(table, ((0, nt - n), (0, 0)))

    ii = jnp.pad(idx_i.astype(jnp.int32), (0, p_pad - p), constant_values=n)
    jj = jnp.pad(idx_j.astype(jnp.int32), (0, p_pad - p), constant_values=n)
    ii = ii.reshape(p_pad // _SUBLEN, _SUBLEN)
    jj = jj.reshape(p_pad // _SUBLEN, _SUBLEN)

    parts = _make_sc_kernel(nt, nchunk)(table, ii, jj)
    parts = parts.reshape(_NC, nt // 128, 128)

    yi0 = jnp.pad(yi[:, 0].astype(jnp.float32), (0, nt - n))
    yi0 = yi0.reshape(nt // 128, 128)

    total = pl.pallas_call(
        _combine_body,
        out_shape=jax.ShapeDtypeStruct((nt // 128, 128), jnp.float32),
    )(parts, yi0)

    return total.reshape(nt)[:n][:, None]


# speedup vs baseline: 1.6923x; 1.6923x over previous
import functools

import jax
import jax.numpy as jnp
from jax import lax
from jax.experimental import pallas as pl
from jax.experimental.pallas import tpu as pltpu
from jax.experimental.pallas import tpu_sc as plsc

_KE = 14.399645
_CUTON = 2.5
_SW_CUTOFF = 7.5
_LR_CUTOFF = 10.0
_CUTOFF = 10.0
_HALF_KE = 0.5 * _KE
_INV_SW_WIDTH = 1.0 / (_SW_CUTOFF - _CUTON)
_INV_LR = 1.0 / _LR_CUTOFF
_INV_LR2 = 1.0 / (_LR_CUTOFF * _LR_CUTOFF)
_INV_CUT = 1.0 / _CUTOFF
_INV_A_CONST = 1.0 / (0.8854 * 0.529177)

_NC = 2
_NS = 16
_NW = _NC * _NS
_LANES = 16
_SUBLEN = 128
_CHUNK = 1024
_NSUB = _CHUNK // _SUBLEN


def _rsqrt(x):
    i = lax.bitcast_convert_type(x, jnp.int32)
    i = jnp.int32(0x5F3759DF) - lax.shift_right_logical(i, 1)
    y = lax.bitcast_convert_type(i, jnp.float32)
    xh = 0.5 * x
    for _ in range(2):
        y = y * (1.5 - xh * y * y)
    return y


def _rcp(x):
    i = lax.bitcast_convert_type(x, jnp.int32)
    i = jnp.int32(0x7EF311C3) - i
    y = lax.bitcast_convert_type(i, jnp.float32)
    for _ in range(2):
        y = y * (2.0 - x * y)
    return y


def _pair_energy(xi, yi_, zi_, qi, c6i, zfi, z23i,
                 xj, yj_, zj_, qj, c6j, zfj, z23j):
    dx = xj - xi
    dy = yj_ - yi_
    dz = zj_ - zi_
    r2 = dx * dx + dy * dy + dz * dz
    u = _rsqrt(r2)
    r = r2 * u
    damped = _rsqrt(r2 + 1.0)
    xs = jnp.clip((r - _CUTON) * _INV_SW_WIDTH, 0.0, 1.0)
    s = (xs * xs * xs) * (xs * (6.0 * xs - 15.0) + 10.0)
    coul = damped + s * (u - damped)
    in_lr = r < _LR_CUTOFF
    shifted = jnp.where(in_lr, coul + r * _INV_LR2 - 2.0 * _INV_LR, 0.0)
    e_c = (qi * qj) * shifted
    xc = r * _INV_CUT
    fc = jnp.where(in_lr,
                   1.0 - (xc * xc * xc) * (xc * (6.0 * xc - 15.0) + 10.0),
                   0.0)
    c6ij = c6i * c6j
    r6 = r2 * r2 * r2
    e_d = (-0.5) * c6ij * _rcp(r6 + 1.0) * fc
    inv_a = (z23i + z23j + 1e-9) * _INV_A_CONST
    xz = r * inv_a
    phi = (0.18175 * jnp.exp(-3.19980 * xz)
           + 0.50986 * jnp.exp(-0.94229 * xz)
           + 0.28022 * jnp.exp(-0.40290 * xz)
           + 0.02817 * jnp.exp(-0.20162 * xz))
    e_z = (zfi * zfj) * u * phi * fc
    return e_c + e_z + e_d


def _make_sc_kernel(nt, nchunk):
    rows_per_tile = nt // _NS
    mesh = plsc.VectorSubcoreMesh(core_axis_name="c", subcore_axis_name="s",
                                  num_cores=_NC, num_subcores=_NS)

    nbuf = 2
    scr = []
    for _ in range(nbuf):
        scr += [
            pltpu.VMEM((_NSUB, _SUBLEN), jnp.int32),
            pltpu.VMEM((_NSUB, _SUBLEN), jnp.int32),
            pltpu.VMEM((_CHUNK, 8), jnp.float32),
            pltpu.VMEM((_CHUNK, 8), jnp.float32),
            pltpu.VMEM((_CHUNK,), jnp.float32),
            pltpu.SemaphoreType.DMA,
            pltpu.SemaphoreType.DMA,
            pltpu.SemaphoreType.DMA,
        ]
    scr += [
        pltpu.VMEM((nt // _NS,), jnp.float32),
        pltpu.VMEM_SHARED((nt,), jnp.float32),
    ]

    @functools.partial(
        pl.kernel,
        out_type=jax.ShapeDtypeStruct((_NC, _NS, rows_per_tile), jnp.float32),
        mesh=mesh,
        scratch_types=scr,
        compiler_params=pltpu.CompilerParams(needs_layout_passes=False,
                                             use_tc_tiling_on_sc=False),
    )
    def sc_kernel(table_hbm, idxi_hbm, idxj_hbm, out_hbm, *bufs):
        buf = [bufs[8 * b:8 * b + 8] for b in range(nbuf)]
        zbuf, acc_sp = bufs[8 * nbuf], bufs[8 * nbuf + 1]
        cid = lax.axis_index("c")
        tid = lax.axis_index("s")

        r0 = tid * rows_per_tile

        def zero_body(i, _):
            zbuf[pl.ds(i * _LANES, _LANES)] = jnp.zeros((_LANES,), jnp.float32)
            return 0
        lax.fori_loop(0, rows_per_tile // _LANES, zero_body, 0)
        pltpu.sync_copy(zbuf, acc_sp.at[pl.ds(r0, rows_per_tile)])
        plsc.subcore_barrier()

        wid = cid * _NS + tid
        base_row = wid * (nchunk * _NSUB)
        lanes = lax.iota(jnp.int32, _LANES)

        def idx_start(b, c):
            row0 = base_row + c * _NSUB
            isem = buf[b][5]
            pltpu.async_copy(idxi_hbm.at[pl.ds(row0, _NSUB)], buf[b][0], isem)
            pltpu.async_copy(idxj_hbm.at[pl.ds(row0, _NSUB)], buf[b][1], isem)

        def idx_wait(b):
            isem = buf[b][5]
            pltpu.make_async_copy(idxi_hbm.at[pl.ds(0, _NSUB)],
                                  buf[b][0], isem).wait()
            pltpu.make_async_copy(idxj_hbm.at[pl.ds(0, _NSUB)],
                                  buf[b][1], isem).wait()

        def gather_start(b):
            idxi_v, idxj_v, rows_i, rows_j = buf[b][:4]
            gsem = buf[b][6]
            for sub in range(_NSUB):
                dst = pl.ds(sub * _SUBLEN, _SUBLEN)
                pltpu.async_copy(table_hbm.at[idxi_v.at[sub]],
                                 rows_i.at[dst], gsem)
                pltpu.async_copy(table_hbm.at[idxj_v.at[sub]],
                                 rows_j.at[dst], gsem)

        def gather_wait(b):
            idxi_v, idxj_v, rows_i, rows_j = buf[b][:4]
            gsem = buf[b][6]
            for sub in range(_NSUB):
                dst = pl.ds(sub * _SUBLEN, _SUBLEN)
                pltpu.make_async_copy(table_hbm.at[idxi_v.at[sub]],
                                      rows_i.at[dst], gsem).wait()
                pltpu.make_async_copy(table_hbm.at[idxj_v.at[sub]],
                                      rows_j.at[dst], gsem).wait()

        def compute(b):
            rows_i, rows_j, evals = buf[b][2], buf[b][3], buf[b][4]

            def pair_body(k, _):
                row16 = k * _LANES + lanes

                def col(ref, ci):
                    return plsc.load_gather(
                        ref, [row16, jnp.full((_LANES,), ci, jnp.int32)])

                e16 = _pair_energy(
                    col(rows_i, 0), col(rows_i, 1), col(rows_i, 2),
                    col(rows_i, 3), col(rows_i, 4), col(rows_i, 5),
                    col(rows_i, 6),
                    col(rows_j, 0), col(rows_j, 1), col(rows_j, 2),
                    col(rows_j, 3), col(rows_j, 4), col(rows_j, 5),
                    col(rows_j, 6))
                evals[pl.ds(k * _LANES, _LANES)] = e16
                return 0

            lax.fori_loop(0, _CHUNK // _LANES, pair_body, 0)

        def scatter_start(b):
            idxi_v, evals, ssem = buf[b][0], buf[b][4], buf[b][7]
            for sub in range(_NSUB):
                pltpu.async_copy(evals.at[pl.ds(sub * _SUBLEN, _SUBLEN)],
                                 acc_sp.at[idxi_v.at[sub]], ssem, add=True)

        def scatter_drain(b):
            idxi_v, evals, ssem = buf[b][0], buf[b][4], buf[b][7]
            for sub in range(_NSUB):
                pltpu.make_async_copy(
                    evals.at[pl.ds(sub * _SUBLEN, _SUBLEN)],
                    acc_sp.at[idxi_v.at[sub]], ssem).wait()

        idx_start(0, 0)
        idx_wait(0)
        gather_start(0)

        def half(b, ob, g, c):
            @pl.when(c + 1 < nchunk)
            def _():
                idx_start(ob, c + 1)
                idx_wait(ob)
                gather_start(ob)

            gather_wait(b)
            compute(b)
            scatter_start(b)
            scatter_drain(b)

        def sched_body(g, _):
            half(0, 1, g, 2 * g)
            half(1, 0, g, 2 * g + 1)
            return 0

        lax.fori_loop(0, nchunk // 2, sched_body, 0)
        plsc.subcore_barrier()
        pltpu.sync_copy(acc_sp.at[pl.ds(r0, rows_per_tile)],
                        out_hbm.at[cid, tid])

    return sc_kernel


def _combine_body(p_ref, y_ref, o_ref):
    o_ref[...] = p_ref[0] + p_ref[1] + y_ref[...]


def kernel(yi, R, partial_charges, c6_table, Z, idx_m, idx_i, idx_j):
    n = Z.shape[0]
    p = idx_i.shape[0]

    nt = ((n + 1 + 2047) // 2048) * 2048
    pairs_per_sweep = _NW * _CHUNK
    nchunk = (p + pairs_per_sweep - 1) // pairs_per_sweep
    nchunk += nchunk % 2
    p_pad = nchunk * pairs_per_sweep

    Z = Z.astype(jnp.int32)
    zf = Z.astype(jnp.float32)
    sc6 = jnp.sqrt(jax.nn.softplus(c6_table.astype(jnp.float32)))
    onehot = (Z[:, None] == jnp.arange(sc6.shape[0], dtype=jnp.int32)[None, :])
    sc6z = jnp.dot(onehot.astype(jnp.float32), sc6)
    sqke = _HALF_KE ** 0.5
    table = jnp.stack(
        [R[:, 0], R[:, 1], R[:, 2],
         partial_charges.astype(jnp.float32) * sqke,
         sc6z, zf * sqke, zf ** 0.23, jnp.zeros((n,), jnp.float32)], axis=1)
    table = jnp.pad(table, ((0, nt - n), (0, 0)))

    ii = jnp.pad(idx_i.astype(jnp.int32), (0, p_pad - p), constant_values=n)
    jj = jnp.pad(idx_j.astype(jnp.int32), (0, p_pad - p), constant_values=n)
    ii = ii.reshape(p_pad // _SUBLEN, _SUBLEN)
    jj = jj.reshape(p_pad // _SUBLEN, _SUBLEN)

    parts = _make_sc_kernel(nt, nchunk)(table, ii, jj)
    parts = parts.reshape(_NC, nt // 128, 128)

    yi0 = jnp.pad(yi[:, 0].astype(jnp.float32), (0, nt - n))
    yi0 = yi0.reshape(nt // 128, 128)

    total = pl.pallas_call(
        _combine_body,
        out_shape=jax.ShapeDtypeStruct((nt // 128, 128), jnp.float32),
    )(parts, yi0)

    return total.reshape(nt)[:n][:, None]


# speedup vs baseline: 2.0128x; 1.1894x over previous
import functools

import jax
import jax.numpy as jnp
from jax import lax
from jax.experimental import pallas as pl
from jax.experimental.pallas import tpu as pltpu
from jax.experimental.pallas import tpu_sc as plsc

_KE = 14.399645
_CUTON = 2.5
_SW_CUTOFF = 7.5
_LR_CUTOFF = 10.0
_CUTOFF = 10.0
_HALF_KE = 0.5 * _KE
_INV_SW_WIDTH = 1.0 / (_SW_CUTOFF - _CUTON)
_INV_LR = 1.0 / _LR_CUTOFF
_INV_LR2 = 1.0 / (_LR_CUTOFF * _LR_CUTOFF)
_INV_CUT = 1.0 / _CUTOFF
_INV_A_CONST = 1.0 / (0.8854 * 0.529177)

_NC = 2
_NS = 16
_NW = _NC * _NS
_LANES = 16
_SUBLEN = 128
_CHUNK = 1024
_NSUB = _CHUNK // _SUBLEN


def _rsqrt(x):
    i = lax.bitcast_convert_type(x, jnp.int32)
    i = jnp.int32(0x5F3759DF) - lax.shift_right_logical(i, 1)
    y = lax.bitcast_convert_type(i, jnp.float32)
    xh = 0.5 * x
    for _ in range(2):
        y = y * (1.5 - xh * y * y)
    return y


def _rcp(x):
    i = lax.bitcast_convert_type(x, jnp.int32)
    i = jnp.int32(0x7EF311C3) - i
    y = lax.bitcast_convert_type(i, jnp.float32)
    for _ in range(2):
        y = y * (2.0 - x * y)
    return y


def _pair_energy(xi, yi_, zi_, qi, c6i, zfi, z23i,
                 xj, yj_, zj_, qj, c6j, zfj, z23j):
    dx = xj - xi
    dy = yj_ - yi_
    dz = zj_ - zi_
    r2 = dx * dx + dy * dy + dz * dz
    u = _rsqrt(r2)
    r = r2 * u
    damped = _rsqrt(r2 + 1.0)
    xs = jnp.clip((r - _CUTON) * _INV_SW_WIDTH, 0.0, 1.0)
    s = (xs * xs * xs) * (xs * (6.0 * xs - 15.0) + 10.0)
    coul = damped + s * (u - damped)
    in_lr = r < _LR_CUTOFF
    shifted = jnp.where(in_lr, coul + r * _INV_LR2 - 2.0 * _INV_LR, 0.0)
    e_c = (qi * qj) * shifted
    xc = r * _INV_CUT
    fc = jnp.where(in_lr,
                   1.0 - (xc * xc * xc) * (xc * (6.0 * xc - 15.0) + 10.0),
                   0.0)
    c6ij = c6i * c6j
    r6 = r2 * r2 * r2
    e_d = (-0.5) * c6ij * _rcp(r6 + 1.0) * fc
    inv_a = (z23i + z23j + 1e-9) * _INV_A_CONST
    xz = r * inv_a
    phi = (0.18175 * jnp.exp(-3.19980 * xz)
           + 0.50986 * jnp.exp(-0.94229 * xz)
           + 0.28022 * jnp.exp(-0.40290 * xz)
           + 0.02817 * jnp.exp(-0.20162 * xz))
    e_z = (zfi * zfj) * u * phi * fc
    return e_c + e_z + e_d


def _make_sc_kernel(nt, nchunk, rows_limit):
    rows_per_tile = nt // _NS
    mesh = plsc.VectorSubcoreMesh(core_axis_name="c", subcore_axis_name="s",
                                  num_cores=_NC, num_subcores=_NS)

    nbuf = 2
    scr = []
    for _ in range(nbuf):
        scr += [
            pltpu.VMEM((_NSUB, _SUBLEN), jnp.int32),
            pltpu.VMEM((_NSUB, _SUBLEN), jnp.int32),
            pltpu.VMEM((_CHUNK, 8), jnp.float32),
            pltpu.VMEM((_CHUNK, 8), jnp.float32),
            pltpu.VMEM((_CHUNK,), jnp.float32),
            pltpu.SemaphoreType.DMA,
            pltpu.SemaphoreType.DMA,
            pltpu.SemaphoreType.DMA,
        ]
    scr += [
        pltpu.VMEM((nt // _NS,), jnp.float32),
        pltpu.VMEM_SHARED((nt,), jnp.float32),
    ]

    @functools.partial(
        pl.kernel,
        out_type=jax.ShapeDtypeStruct((_NC, _NS, rows_per_tile), jnp.float32),
        mesh=mesh,
        scratch_types=scr,
        compiler_params=pltpu.CompilerParams(needs_layout_passes=False,
                                             use_tc_tiling_on_sc=False),
    )
    def sc_kernel(table_hbm, idxi_hbm, idxj_hbm, out_hbm, *bufs):
        buf = [bufs[8 * b:8 * b + 8] for b in range(nbuf)]
        zbuf, acc_sp = bufs[8 * nbuf], bufs[8 * nbuf + 1]
        cid = lax.axis_index("c")
        tid = lax.axis_index("s")

        r0 = tid * rows_per_tile

        def zero_body(i, _):
            zbuf[pl.ds(i * _LANES, _LANES)] = jnp.zeros((_LANES,), jnp.float32)
            return 0
        lax.fori_loop(0, rows_per_tile // _LANES, zero_body, 0)
        pltpu.sync_copy(zbuf, acc_sp.at[pl.ds(r0, rows_per_tile)])
        plsc.subcore_barrier()

        wid = cid * _NS + tid
        base_row = wid * (nchunk * _NSUB)
        nvalid = jnp.clip((rows_limit - base_row) // _NSUB, 0, nchunk)
        lanes = lax.iota(jnp.int32, _LANES)

        def idx_start(b, c):
            row0 = base_row + c * _NSUB
            isem = buf[b][5]
            pltpu.async_copy(idxi_hbm.at[pl.ds(row0, _NSUB)], buf[b][0], isem)
            pltpu.async_copy(idxj_hbm.at[pl.ds(row0, _NSUB)], buf[b][1], isem)

        def idx_wait(b):
            isem = buf[b][5]
            pltpu.make_async_copy(idxi_hbm.at[pl.ds(0, _NSUB)],
                                  buf[b][0], isem).wait()
            pltpu.make_async_copy(idxj_hbm.at[pl.ds(0, _NSUB)],
                                  buf[b][1], isem).wait()

        def gather_start(b):
            idxi_v, idxj_v, rows_i, rows_j = buf[b][:4]
            gsem = buf[b][6]
            for sub in range(_NSUB):
                dst = pl.ds(sub * _SUBLEN, _SUBLEN)
                pltpu.async_copy(table_hbm.at[idxi_v.at[sub]],
                                 rows_i.at[dst], gsem)
                pltpu.async_copy(table_hbm.at[idxj_v.at[sub]],
                                 rows_j.at[dst], gsem)

        def gather_wait(b):
            idxi_v, idxj_v, rows_i, rows_j = buf[b][:4]
            gsem = buf[b][6]
            for sub in range(_NSUB):
                dst = pl.ds(sub * _SUBLEN, _SUBLEN)
                pltpu.make_async_copy(table_hbm.at[idxi_v.at[sub]],
                                      rows_i.at[dst], gsem).wait()
                pltpu.make_async_copy(table_hbm.at[idxj_v.at[sub]],
                                      rows_j.at[dst], gsem).wait()

        def compute(b):
            rows_i, rows_j, evals = buf[b][2], buf[b][3], buf[b][4]

            def pair_body(k, _):
                row16 = k * _LANES + lanes

                def col(ref, ci):
                    return plsc.load_gather(
                        ref, [row16, jnp.full((_LANES,), ci, jnp.int32)])

                e16 = _pair_energy(
                    col(rows_i, 0), col(rows_i, 1), col(rows_i, 2),
                    col(rows_i, 3), col(rows_i, 4), col(rows_i, 5),
                    col(rows_i, 6),
                    col(rows_j, 0), col(rows_j, 1), col(rows_j, 2),
                    col(rows_j, 3), col(rows_j, 4), col(rows_j, 5),
                    col(rows_j, 6))
                evals[pl.ds(k * _LANES, _LANES)] = e16
                return 0

            lax.fori_loop(0, _CHUNK // _LANES, pair_body, 0)

        def scatter_start(b):
            idxi_v, evals, ssem = buf[b][0], buf[b][4], buf[b][7]
            for sub in range(_NSUB):
                pltpu.async_copy(evals.at[pl.ds(sub * _SUBLEN, _SUBLEN)],
                                 acc_sp.at[idxi_v.at[sub]], ssem, add=True)

        def scatter_drain(b):
            idxi_v, evals, ssem = buf[b][0], buf[b][4], buf[b][7]
            for sub in range(_NSUB):
                pltpu.make_async_copy(
                    evals.at[pl.ds(sub * _SUBLEN, _SUBLEN)],
                    acc_sp.at[idxi_v.at[sub]], ssem).wait()

        @pl.when(nvalid > 0)
        def _():
            idx_start(0, 0)
            idx_wait(0)
            gather_start(0)

        def half(b, ob, g, c):
            @pl.when(c + 1 < nvalid)
            def _():
                idx_start(ob, c + 1)
                idx_wait(ob)
                gather_start(ob)

            @pl.when(c < nvalid)
            def _():
                gather_wait(b)
                compute(b)
                scatter_start(b)
                scatter_drain(b)

        def sched_body(g, _):
            half(0, 1, g, 2 * g)
            half(1, 0, g, 2 * g + 1)
            return 0

        lax.fori_loop(0, nchunk // 2, sched_body, 0)
        plsc.subcore_barrier()
        pltpu.sync_copy(acc_sp.at[pl.ds(r0, rows_per_tile)],
                        out_hbm.at[cid, tid])

    return sc_kernel


def _combine_body(p_ref, y_ref, o_ref):
    o_ref[...] = p_ref[0] + p_ref[1] + y_ref[...]


def kernel(yi, R, partial_charges, c6_table, Z, idx_m, idx_i, idx_j):
    n = Z.shape[0]
    p = idx_i.shape[0]

    nt = ((n + 1 + 2047) // 2048) * 2048
    pairs_per_sweep = _NW * _CHUNK
    nchunk = (p + pairs_per_sweep - 1) // pairs_per_sweep
    nchunk += nchunk % 2

    Z = Z.astype(jnp.int32)
    zf = Z.astype(jnp.float32)
    sc6 = jnp.sqrt(jax.nn.softplus(c6_table.astype(jnp.float32)))
    onehot = (Z[:, None] == jnp.arange(sc6.shape[0], dtype=jnp.int32)[None, :])
    sc6z = jnp.dot(onehot.astype(jnp.float32), sc6)
    sqke = _HALF_KE ** 0.5
    table = jnp.stack(
        [R[:, 0], R[:, 1], R[:, 2],
         partial_charges.astype(jnp.float32) * sqke,
         sc6z, zf * sqke, zf ** 0.23, jnp.zeros((n,), jnp.float32)], axis=1)

    ii = idx_i.astype(jnp.int32)
    jj = idx_j.astype(jnp.int32)
    if p % _CHUNK != 0:
        p_use = ((p + _CHUNK - 1) // _CHUNK) * _CHUNK
        ii = jnp.pad(ii, (0, p_use - p), constant_values=n)
        jj = jnp.pad(jj, (0, p_use - p), constant_values=n)
        table = jnp.pad(table, ((0, 1), (0, 0)))
    else:
        p_use = p
    ii = ii.reshape(p_use // _SUBLEN, _SUBLEN)
    jj = jj.reshape(p_use // _SUBLEN, _SUBLEN)

    parts = _make_sc_kernel(nt, nchunk, p_use // _SUBLEN)(table, ii, jj)
    parts = parts.reshape(_NC, nt // 128, 128)

    yi0 = jnp.pad(yi[:, 0].astype(jnp.float32), (0, nt - n))
    yi0 = yi0.reshape(nt // 128, 128)

    total = pl.pallas_call(
        _combine_body,
        out_shape=jax.ShapeDtypeStruct((nt // 128, 128), jnp.float32),
    )(parts, yi0)

    return total.reshape(nt)[:n][:, None]
